# baseline (speedup 1.0000x reference)
import functools
import math

import jax
import jax.numpy as jnp
from jax import lax
from jax.experimental import pallas as pl
from jax.experimental.pallas import tpu as pltpu
from jax.experimental.pallas import tpu_sc as plsc

_EPS = 1e-3
_ISQ = 1.0 / math.sqrt(1.0 + _EPS)

_NC = 2
_NS = 16
_NW = _NC * _NS
_C = 128
_K = 4
_KR = 2
_DEGW = 16


@functools.lru_cache(maxsize=None)
def _make_deg_kernel(n_chunks, N, n_pad):
    mesh = plsc.VectorSubcoreMesh(core_axis_name="c", subcore_axis_name="s")
    zr = 64
    rows_per_zero = n_pad // _NS
    n_zero = rows_per_zero // zr
    rows_per_out = n_pad // _NS

    @functools.partial(
        pl.kernel,
        out_type=jax.ShapeDtypeStruct((_NC, n_pad, _DEGW), jnp.float32),
        mesh=mesh,
        scratch_types=[
            pltpu.VMEM((2, _C), jnp.int32),
            pltpu.VMEM((_C, _DEGW), jnp.float32),
            pltpu.VMEM((zr, _DEGW), jnp.float32),
            pltpu.VMEM_SHARED((n_pad, _DEGW), jnp.float32),
        ],
    )
    def k(sd_hbm, out_hbm, idxb, ones_v, zbuf, acc):
        c = lax.axis_index("c")
        s = lax.axis_index("s")
        wid = c * _NS + s
        one = jnp.ones((16,), jnp.float32)
        zero = jnp.zeros((16,), jnp.float32)

        def fill(i, carry):
            ones_v[i, :] = one
            return carry

        lax.fori_loop(0, _C, fill, 0)

        def zfill(i, carry):
            zbuf[i, :] = zero
            return carry

        lax.fori_loop(0, zr, zfill, 0)

        zbase = s * rows_per_zero

        def zero_body(i, carry):
            pltpu.sync_copy(zbuf, acc.at[pl.ds(zbase + i * zr, zr)])
            return carry

        lax.fori_loop(0, n_zero, zero_body, 0)
        plsc.subcore_barrier()

        cbase = wid * (n_chunks + _K)

        def body(i, carry):
            pltpu.sync_copy(sd_hbm.at[cbase + i], idxb)
            pltpu.sync_copy(ones_v, acc.at[idxb.at[1]], add=True)
            return carry

        lax.fori_loop(0, n_chunks, body, 0)
        plsc.subcore_barrier()

        r0 = s * rows_per_out
        pltpu.sync_copy(acc.at[pl.ds(r0, rows_per_out)],
                        out_hbm.at[c, pl.ds(r0, rows_per_out)])

    return k


@functools.lru_cache(maxsize=None)
def _make_edge_kernel(n_chunks, N, n_pad, D):
    mesh = plsc.VectorSubcoreMesh(core_axis_name="c", subcore_axis_name="s")
    zr = 8
    rows_per_zero = n_pad // _NS
    n_zero = rows_per_zero // zr
    rows_per_out = n_pad // _NS

    @functools.partial(
        pl.kernel,
        out_type=jax.ShapeDtypeStruct((_NC, n_pad, D), jnp.float32),
        mesh=mesh,
        scratch_types=[
            pltpu.VMEM((_K, 2, _C), jnp.int32),
            pltpu.VMEM((_KR, _C, D), jnp.float32),
            pltpu.VMEM((zr, D), jnp.float32),
            pltpu.VMEM_SHARED((n_pad, D), jnp.float32),
            pltpu.SemaphoreType.DMA((_K,)),
            pltpu.SemaphoreType.DMA((_KR,)),
            pltpu.SemaphoreType.DMA((_KR,)),
        ],
    )
    def k(sd_hbm, q_hbm, out_hbm, idxb, rows, zbuf, acc, isem, gsem, ssem):
        c = lax.axis_index("c")
        s = lax.axis_index("s")
        wid = c * _NS + s
        zero = jnp.zeros((16,), jnp.float32)
        for i in range(zr):
            for j in range(D // 16):
                zbuf[i, pl.ds(j * 16, 16)] = zero

        zbase = s * rows_per_zero

        def zero_body(i, carry):
            pltpu.sync_copy(zbuf, acc.at[pl.ds(zbase + i * zr, zr)])
            return carry

        lax.fori_loop(0, n_zero, zero_body, 0)
        plsc.subcore_barrier()

        cbase = wid * (n_chunks + _K)

        def idx_load(chunk, sl):
            pltpu.async_copy(sd_hbm.at[cbase + chunk], idxb.at[sl],
                             isem.at[sl])

        def idx_wait(sl):
            pltpu.make_async_copy(sd_hbm.at[cbase], idxb.at[sl],
                                  isem.at[sl]).wait()

        def gather(isl, rsl):
            pltpu.async_copy(q_hbm.at[idxb.at[isl, 0]], rows.at[rsl],
                             gsem.at[rsl])

        def gather_wait(rsl):
            pltpu.make_async_copy(q_hbm.at[idxb.at[0, 0]], rows.at[rsl],
                                  gsem.at[rsl]).wait()

        def scatter_start(isl, rsl):
            return pltpu.async_copy(rows.at[rsl], acc.at[idxb.at[isl, 1]],
                                    ssem.at[rsl], add=True)

        def step(c, c_dyn):
            isl, rsl = c % _K, c % _KR
            gather_wait((c - 1) % _KR)
            d = scatter_start((c - 1) % _K, (c - 1) % _KR)
            idx_wait(isl)
            gather(isl, rsl)
            idx_load(c_dyn + 2, (c + 2) % _K)
            d.wait()

        idx_load(0, 0)
        idx_load(1, 1)
        idx_wait(0)
        gather(0, 0)
        idx_load(2, 2)
        step(1, 1)

        n_body = (n_chunks - 2) // 4

        def body(u, carry):
            c0 = 2 + 4 * u
            for t in range(4):
                step(2 + t, c0 + t)
            return carry

        lax.fori_loop(0, n_body, body, 0)

        for c in range(2 + 4 * n_body, n_chunks):
            step(c, c)
        gather_wait((n_chunks - 1) % _KR)
        scatter_start((n_chunks - 1) % _K, (n_chunks - 1) % _KR).wait()
        idx_wait(n_chunks % _K)
        idx_wait((n_chunks + 1) % _K)
        plsc.subcore_barrier()

        r0 = s * rows_per_out
        pltpu.sync_copy(acc.at[pl.ds(r0, rows_per_out)],
                        out_hbm.at[c, pl.ds(r0, rows_per_out)])

    return k


_R = 512


def _first_tc(x, W, dA, dB):
    N, D = x.shape
    grid = pl.cdiv(N, _R)

    def body(x_ref, w_ref, da_ref, db_ref, q_ref, dis_ref):
        dis = lax.rsqrt(da_ref[...] + db_ref[...] + 1.0)
        q_ref[...] = dis * jnp.dot(x_ref[...], w_ref[...],
                                   preferred_element_type=jnp.float32)
        dis_ref[...] = dis

    return pl.pallas_call(
        body,
        grid=(grid,),
        in_specs=[
            pl.BlockSpec((_R, D), lambda i: (i, 0)),
            pl.BlockSpec((D, D), lambda i: (0, 0)),
            pl.BlockSpec((_R, 1), lambda i: (i, 0)),
            pl.BlockSpec((_R, 1), lambda i: (i, 0)),
        ],
        out_specs=[
            pl.BlockSpec((_R, D), lambda i: (i, 0)),
            pl.BlockSpec((_R, 1), lambda i: (i, 0)),
        ],
        out_shape=[
            jax.ShapeDtypeStruct((N, D), jnp.float32),
            jax.ShapeDtypeStruct((N, 1), jnp.float32),
        ],
    )(x, W, dA, dB)


def _mid_tc(s, qp, dis, g, b, be, W):
    N, D = qp.shape

    grid = pl.cdiv(N, _R)

    def body(sa_ref, sb_ref, qp_ref, dis_ref, g_ref, b_ref, be_ref, w_ref,
             q_ref):
        d = dis_ref[...]
        agg = d * (sa_ref[0] + sb_ref[0] + qp_ref[...])
        gs = g_ref[...] * _ISQ
        h = jnp.maximum(gs * agg + (gs * b_ref[...] + be_ref[...]), 0.0)
        q_ref[...] = d * jnp.dot(h, w_ref[...],
                                 preferred_element_type=jnp.float32)

    return pl.pallas_call(
        body,
        grid=(grid,),
        in_specs=[
            pl.BlockSpec((1, _R, D), lambda i: (0, i, 0)),
            pl.BlockSpec((1, _R, D), lambda i: (1, i, 0)),
            pl.BlockSpec((_R, D), lambda i: (i, 0)),
            pl.BlockSpec((_R, 1), lambda i: (i, 0)),
            pl.BlockSpec((1, D), lambda i: (0, 0)),
            pl.BlockSpec((1, D), lambda i: (0, 0)),
            pl.BlockSpec((1, D), lambda i: (0, 0)),
            pl.BlockSpec((D, D), lambda i: (0, 0)),
        ],
        out_specs=pl.BlockSpec((_R, D), lambda i: (i, 0)),
        out_shape=jax.ShapeDtypeStruct((N, D), jnp.float32),
    )(s, s, qp, dis, g, b, be, W)


def _last_tc(s, qp, dis, g, b, be):
    N, D = qp.shape

    grid = pl.cdiv(N, _R)

    def body(sa_ref, sb_ref, qp_ref, dis_ref, g_ref, b_ref, be_ref, o_ref):
        d = dis_ref[...]
        agg = d * (sa_ref[0] + sb_ref[0] + qp_ref[...])
        gs = g_ref[...] * _ISQ
        o_ref[...] = jnp.maximum(gs * agg + (gs * b_ref[...] + be_ref[...]),
                                 0.0)

    return pl.pallas_call(
        body,
        grid=(grid,),
        in_specs=[
            pl.BlockSpec((1, _R, D), lambda i: (0, i, 0)),
            pl.BlockSpec((1, _R, D), lambda i: (1, i, 0)),
            pl.BlockSpec((_R, D), lambda i: (i, 0)),
            pl.BlockSpec((_R, 1), lambda i: (i, 0)),
            pl.BlockSpec((1, D), lambda i: (0, 0)),
            pl.BlockSpec((1, D), lambda i: (0, 0)),
            pl.BlockSpec((1, D), lambda i: (0, 0)),
        ],
        out_specs=pl.BlockSpec((_R, D), lambda i: (i, 0)),
        out_shape=jax.ShapeDtypeStruct((N, D), jnp.float32),
    )(s, s, qp, dis, g, b, be)


def kernel(x, edge_index, W1, b1, g1, be1, W2, b2, g2, be2, W3, b3, g3, be3):
    N, D = x.shape
    E = edge_index.shape[1]
    assert D % 16 == 0 and N % _NS == 0

    n_super = pl.cdiv(E, _NW * _C * _K)
    n_chunks = n_super * _K
    E_pad = n_chunks * _NW * _C
    n_pad = ((N + 1 + 127) // 128) * 128

    src = edge_index[0].astype(jnp.int32)
    dst = edge_index[1].astype(jnp.int32)
    pad = E_pad - E
    if pad:
        src = jnp.concatenate([src, jnp.zeros((pad,), jnp.int32)])
        spread = N + jnp.arange(pad, dtype=jnp.int32) % (n_pad - N)
        dst = jnp.concatenate([dst, spread])
    sd = jnp.stack([src.reshape(_NW, n_chunks, _C),
                    dst.reshape(_NW, n_chunks, _C)], axis=2)
    tail = jnp.full((_NW, _K, 2, _C), 0, jnp.int32)
    tail = tail.at[:, :, 1, :].set(N)
    sd = jnp.concatenate([sd, tail], axis=1).reshape(-1, 2, _C)

    deg = _make_deg_kernel(n_chunks, N, n_pad)(sd)
    dA = deg[0, :N, :1]
    dB = deg[1, :N, :1]

    edge = _make_edge_kernel(n_chunks, N, n_pad, D)

    q1, dis = _first_tc(x, W1, dA, dB)
    s1 = edge(sd, q1)
    q2 = _mid_tc(s1, q1, dis, g1.reshape(1, D), b1.reshape(1, D),
                 be1.reshape(1, D), W2)
    s2 = edge(sd, q2)
    q3 = _mid_tc(s2, q2, dis, g2.reshape(1, D), b2.reshape(1, D),
                 be2.reshape(1, D), W3)
    s3 = edge(sd, q3)
    return _last_tc(s3, q3, dis, g3.reshape(1, D), b3.reshape(1, D),
                    be3.reshape(1, D))


# speedup vs baseline: 1.5492x; 1.5492x over previous
import functools
import math

import jax
import jax.numpy as jnp
from jax import lax
from jax.experimental import pallas as pl
from jax.experimental.pallas import tpu as pltpu
from jax.experimental.pallas import tpu_sc as plsc

_EPS = 1e-3
_ISQ = 1.0 / math.sqrt(1.0 + _EPS)

_NC = 2
_NS = 16
_NW = _NC * _NS
_C = 128
_K = 4
_KR = 2
_DEGW = 16


@functools.lru_cache(maxsize=None)
def _make_deg_kernel(n_chunks, N, n_pad):
    mesh = plsc.VectorSubcoreMesh(core_axis_name="c", subcore_axis_name="s")
    zr = 64
    rows_per_zero = n_pad // _NS
    n_zero = rows_per_zero // zr
    rows_per_out = n_pad // _NS

    @functools.partial(
        pl.kernel,
        out_type=jax.ShapeDtypeStruct((_NC, n_pad, _DEGW), jnp.float32),
        mesh=mesh,
        scratch_types=[
            pltpu.VMEM((2, _C), jnp.int32),
            pltpu.VMEM((_C, _DEGW), jnp.float32),
            pltpu.VMEM((zr, _DEGW), jnp.float32),
            pltpu.VMEM_SHARED((n_pad, _DEGW), jnp.float32),
        ],
    )
    def k(sd_hbm, out_hbm, idxb, ones_v, zbuf, acc):
        c = lax.axis_index("c")
        s = lax.axis_index("s")
        wid = c * _NS + s
        one = jnp.ones((16,), jnp.float32)
        zero = jnp.zeros((16,), jnp.float32)

        def fill(i, carry):
            ones_v[i, :] = one
            return carry

        lax.fori_loop(0, _C, fill, 0)

        def zfill(i, carry):
            zbuf[i, :] = zero
            return carry

        lax.fori_loop(0, zr, zfill, 0)

        zbase = s * rows_per_zero

        def zero_body(i, carry):
            pltpu.sync_copy(zbuf, acc.at[pl.ds(zbase + i * zr, zr)])
            return carry

        lax.fori_loop(0, n_zero, zero_body, 0)
        plsc.subcore_barrier()

        cbase = wid * (n_chunks + _K)

        def body(i, carry):
            pltpu.sync_copy(sd_hbm.at[cbase + i], idxb)
            pltpu.sync_copy(ones_v, acc.at[idxb.at[1]], add=True)
            return carry

        lax.fori_loop(0, n_chunks, body, 0)
        plsc.subcore_barrier()

        r0 = s * rows_per_out
        pltpu.sync_copy(acc.at[pl.ds(r0, rows_per_out)],
                        out_hbm.at[c, pl.ds(r0, rows_per_out)])

    return k


@functools.lru_cache(maxsize=None)
def _make_edge_kernel(n_chunks, N, n_pad, D):
    mesh = plsc.VectorSubcoreMesh(core_axis_name="c", subcore_axis_name="s")
    zr = 8
    rows_per_zero = n_pad // _NS
    n_zero = rows_per_zero // zr
    rows_per_out = n_pad // _NS

    @functools.partial(
        pl.kernel,
        out_type=jax.ShapeDtypeStruct((_NC, n_pad, D), jnp.float32),
        mesh=mesh,
        scratch_types=[
            pltpu.VMEM((1, 2, _C), jnp.int32),
            pltpu.VMEM((1, _C, D), jnp.float32),
            pltpu.VMEM((zr, D), jnp.float32),
            pltpu.VMEM_SHARED((n_pad, D), jnp.float32),
            pltpu.SemaphoreType.DMA((1,)),
        ],
    )
    def k(sd_hbm, q_hbm, out_hbm, idxb, rows, zbuf, acc, gsem):
        c = lax.axis_index("c")
        s = lax.axis_index("s")
        wid = c * _NS + s
        zero = jnp.zeros((16,), jnp.float32)
        for i in range(zr):
            for j in range(D // 16):
                zbuf[i, pl.ds(j * 16, 16)] = zero

        zbase = s * rows_per_zero

        def zero_body(i, carry):
            pltpu.sync_copy(zbuf, acc.at[pl.ds(zbase + i * zr, zr)])
            return carry

        lax.fori_loop(0, n_zero, zero_body, 0)
        plsc.subcore_barrier()

        cbase = wid * (n_chunks + _K)

        def body(i, carry):
            pltpu.sync_copy(sd_hbm.at[cbase + i], idxb.at[0])
            pltpu.async_copy(q_hbm.at[idxb.at[0, 0]], rows.at[0],
                             gsem.at[0]).wait()
            pltpu.sync_copy(rows.at[0], acc.at[idxb.at[0, 1]], add=True)
            return carry

        lax.fori_loop(0, n_chunks, body, 0)
        plsc.subcore_barrier()

        r0 = s * rows_per_out
        pltpu.sync_copy(acc.at[pl.ds(r0, rows_per_out)],
                        out_hbm.at[c, pl.ds(r0, rows_per_out)])

    return k


_R = 512


def _first_tc(x, W, dA, dB):
    N, D = x.shape
    grid = pl.cdiv(N, _R)

    def body(x_ref, w_ref, da_ref, db_ref, q_ref, dis_ref):
        dis = lax.rsqrt(da_ref[...] + db_ref[...] + 1.0)
        q_ref[...] = dis * jnp.dot(x_ref[...], w_ref[...],
                                   preferred_element_type=jnp.float32)
        dis_ref[...] = dis

    return pl.pallas_call(
        body,
        grid=(grid,),
        in_specs=[
            pl.BlockSpec((_R, D), lambda i: (i, 0)),
            pl.BlockSpec((D, D), lambda i: (0, 0)),
            pl.BlockSpec((_R, 1), lambda i: (i, 0)),
            pl.BlockSpec((_R, 1), lambda i: (i, 0)),
        ],
        out_specs=[
            pl.BlockSpec((_R, D), lambda i: (i, 0)),
            pl.BlockSpec((_R, 1), lambda i: (i, 0)),
        ],
        out_shape=[
            jax.ShapeDtypeStruct((N, D), jnp.float32),
            jax.ShapeDtypeStruct((N, 1), jnp.float32),
        ],
    )(x, W, dA, dB)


def _mid_tc(s, qp, dis, g, b, be, W):
    N, D = qp.shape

    grid = pl.cdiv(N, _R)

    def body(sa_ref, sb_ref, qp_ref, dis_ref, g_ref, b_ref, be_ref, w_ref,
             q_ref):
        d = dis_ref[...]
        agg = d * (sa_ref[0] + sb_ref[0] + qp_ref[...])
        gs = g_ref[...] * _ISQ
        h = jnp.maximum(gs * agg + (gs * b_ref[...] + be_ref[...]), 0.0)
        q_ref[...] = d * jnp.dot(h, w_ref[...],
                                 preferred_element_type=jnp.float32)

    return pl.pallas_call(
        body,
        grid=(grid,),
        in_specs=[
            pl.BlockSpec((1, _R, D), lambda i: (0, i, 0)),
            pl.BlockSpec((1, _R, D), lambda i: (1, i, 0)),
            pl.BlockSpec((_R, D), lambda i: (i, 0)),
            pl.BlockSpec((_R, 1), lambda i: (i, 0)),
            pl.BlockSpec((1, D), lambda i: (0, 0)),
            pl.BlockSpec((1, D), lambda i: (0, 0)),
            pl.BlockSpec((1, D), lambda i: (0, 0)),
            pl.BlockSpec((D, D), lambda i: (0, 0)),
        ],
        out_specs=pl.BlockSpec((_R, D), lambda i: (i, 0)),
        out_shape=jax.ShapeDtypeStruct((N, D), jnp.float32),
    )(s, s, qp, dis, g, b, be, W)


def _last_tc(s, qp, dis, g, b, be):
    N, D = qp.shape

    grid = pl.cdiv(N, _R)

    def body(sa_ref, sb_ref, qp_ref, dis_ref, g_ref, b_ref, be_ref, o_ref):
        d = dis_ref[...]
        agg = d * (sa_ref[0] + sb_ref[0] + qp_ref[...])
        gs = g_ref[...] * _ISQ
        o_ref[...] = jnp.maximum(gs * agg + (gs * b_ref[...] + be_ref[...]),
                                 0.0)

    return pl.pallas_call(
        body,
        grid=(grid,),
        in_specs=[
            pl.BlockSpec((1, _R, D), lambda i: (0, i, 0)),
            pl.BlockSpec((1, _R, D), lambda i: (1, i, 0)),
            pl.BlockSpec((_R, D), lambda i: (i, 0)),
            pl.BlockSpec((_R, 1), lambda i: (i, 0)),
            pl.BlockSpec((1, D), lambda i: (0, 0)),
            pl.BlockSpec((1, D), lambda i: (0, 0)),
            pl.BlockSpec((1, D), lambda i: (0, 0)),
        ],
        out_specs=pl.BlockSpec((_R, D), lambda i: (i, 0)),
        out_shape=jax.ShapeDtypeStruct((N, D), jnp.float32),
    )(s, s, qp, dis, g, b, be)


def kernel(x, edge_index, W1, b1, g1, be1, W2, b2, g2, be2, W3, b3, g3, be3):
    N, D = x.shape
    E = edge_index.shape[1]
    assert D % 16 == 0 and N % _NS == 0

    n_super = pl.cdiv(E, _NW * _C * _K)
    n_chunks = n_super * _K
    E_pad = n_chunks * _NW * _C
    n_pad = ((N + 1 + 127) // 128) * 128

    src = edge_index[0].astype(jnp.int32)
    dst = edge_index[1].astype(jnp.int32)
    pad = E_pad - E
    if pad:
        src = jnp.concatenate([src, jnp.zeros((pad,), jnp.int32)])
        spread = N + jnp.arange(pad, dtype=jnp.int32) % (n_pad - N)
        dst = jnp.concatenate([dst, spread])
    sd = jnp.stack([src.reshape(_NW, n_chunks, _C),
                    dst.reshape(_NW, n_chunks, _C)], axis=2)
    tail = jnp.full((_NW, _K, 2, _C), 0, jnp.int32)
    tail = tail.at[:, :, 1, :].set(N)
    sd = jnp.concatenate([sd, tail], axis=1).reshape(-1, 2, _C)

    deg = _make_deg_kernel(n_chunks, N, n_pad)(sd)
    dA = deg[0, :N, :1]
    dB = deg[1, :N, :1]

    edge = _make_edge_kernel(n_chunks, N, n_pad, D)

    q1, dis = _first_tc(x, W1, dA, dB)
    s1 = edge(sd, q1)
    q2 = _mid_tc(s1, q1, dis, g1.reshape(1, D), b1.reshape(1, D),
                 be1.reshape(1, D), W2)
    s2 = edge(sd, q2)
    q3 = _mid_tc(s2, q2, dis, g2.reshape(1, D), b2.reshape(1, D),
                 be2.reshape(1, D), W3)
    s3 = edge(sd, q3)
    return _last_tc(s3, q3, dis, g3.reshape(1, D), b3.reshape(1, D),
                    be3.reshape(1, D))
